# baseline (speedup 1.0000x reference)
import functools

import jax
import jax.numpy as jnp
from jax import lax
from jax.experimental import pallas as pl
from jax.experimental.pallas import tpu as pltpu
from jax.experimental.pallas import tpu_sc as plsc

DIM = 64
B = 4096
L = 200
NW = 32
TOTAL = B * L
PER_W = TOTAL // NW
CHUNK = 320
NCHUNKS = PER_W // CHUNK
IDXW = 128
NPAIR = NCHUNKS // 2


def _body(x_hbm, lens_hbm, table_hbm, out_hbm,
          idx0, idx1, rin0, rin1, rout0, rout1, mask_v, lens_v,
          gsem0, gsem1, ssem0, ssem1):
    idx = (idx0, idx1)
    rin = (rin0, rin1)
    rout = (rout0, rout1)
    gsem = (gsem0, gsem1)
    ssem = (ssem0, ssem1)

    c_ax = lax.axis_index("c")
    s_ax = lax.axis_index("s")
    wid = s_ax * 2 + c_ax
    base = wid * PER_W

    pltpu.sync_copy(lens_hbm, lens_v)

    lane = lax.iota(jnp.int32, 16)

    def stage_and_fire(c, b):
        pltpu.sync_copy(x_hbm.at[pl.ds(base + c * CHUNK, CHUNK)], idx[b])
        for j in range(0, CHUNK, IDXW):
            w = min(IDXW, CHUNK - j)
            pltpu.async_copy(
                table_hbm.at[idx[b].at[pl.ds(j, w)]],
                rin[b].at[pl.ds(j, w)],
                gsem[b],
            )

    def compute(c, b, bl):
        def mask_body(i, bl):
            b_s, l_s = bl
            lvec = l_s + lane
            wrap = jnp.where(lvec >= L, 1, 0)
            bvec = b_s + wrap
            pos = lvec - wrap * L
            lv = plsc.load_gather(lens_v, [bvec])
            mask_v[pl.ds(i * 16, 16)] = jnp.where(pos < lv, 1.0, 0.0)
            l_n = l_s + 16
            w = jnp.where(l_n >= L, 1, 0)
            return (b_s + w, l_n - w * L)

        bl = lax.fori_loop(0, CHUNK // 16, mask_body, bl)

        def row_body(r, _):
            m = plsc.load_gather(mask_v, [jnp.full((16,), r, dtype=jnp.int32)])
            for j in range(DIM // 16):
                d = rin[b][r, pl.ds(j * 16, 16)]
                rout[b][r, pl.ds(j * 16, 16)] = jnp.maximum(d, 0.0) * m
            return 0

        lax.fori_loop(0, CHUNK, row_body, 0, unroll=4)
        return bl

    def fire_scatter(c, b):
        pltpu.async_copy(rout[b], out_hbm.at[pl.ds(base + c * CHUNK, CHUNK)], ssem[b])

    def drain_scatter(b):
        pltpu.make_async_copy(rout[b], out_hbm.at[pl.ds(base, CHUNK)], ssem[b]).wait()

    def chunk_step(c, b, bl, first, fire_next):
        if fire_next is not None:
            stage_and_fire(fire_next, 1 - b)
        pltpu.make_async_copy(
            table_hbm.at[idx[b].at[pl.ds(0, CHUNK)]], rin[b], gsem[b]
        ).wait()
        if not first:
            drain_scatter(b)
        bl = compute(c, b, bl)
        fire_scatter(c, b)
        return bl

    stage_and_fire(0, 0)
    bl = (wid * (PER_W // L), jnp.int32(0))

    bl = chunk_step(0, 0, bl, True, 1)
    bl = chunk_step(1, 1, bl, True, 2)

    def pair_body(q, bl):
        c0 = 2 * q
        bl = chunk_step(c0, 0, bl, False, c0 + 1)
        bl = chunk_step(c0 + 1, 1, bl, False, c0 + 2)
        return bl

    bl = lax.fori_loop(1, NPAIR - 1, pair_body, bl)

    c0 = NCHUNKS - 2
    bl = chunk_step(c0, 0, bl, False, c0 + 1)
    bl = chunk_step(c0 + 1, 1, bl, False, None)

    drain_scatter(0)
    drain_scatter(1)


@jax.jit
def _run(xf, x_lens, table):
    mesh = plsc.VectorSubcoreMesh(core_axis_name="c", subcore_axis_name="s")
    k = functools.partial(
        pl.kernel,
        mesh=mesh,
        out_type=jax.ShapeDtypeStruct((TOTAL, DIM), jnp.float32),
        scratch_types=[
            pltpu.VMEM((CHUNK,), jnp.int32),
            pltpu.VMEM((CHUNK,), jnp.int32),
            pltpu.VMEM((CHUNK, DIM), jnp.float32),
            pltpu.VMEM((CHUNK, DIM), jnp.float32),
            pltpu.VMEM((CHUNK, DIM), jnp.float32),
            pltpu.VMEM((CHUNK, DIM), jnp.float32),
            pltpu.VMEM((CHUNK,), jnp.float32),
            pltpu.VMEM((B,), jnp.int32),
            pltpu.SemaphoreType.DMA,
            pltpu.SemaphoreType.DMA,
            pltpu.SemaphoreType.DMA,
            pltpu.SemaphoreType.DMA,
        ],
        compiler_params=pltpu.CompilerParams(
            use_tc_tiling_on_sc=False, needs_layout_passes=False
        ),
    )(_body)
    return k(xf, x_lens, table)


def kernel(x, x_lens, table):
    xf = x.reshape(TOTAL)
    out = _run(xf, x_lens, table)
    return out.reshape(B, L, DIM)
